# baseline (speedup 1.0000x reference)
import functools

import jax
import jax.numpy as jnp
from jax import lax
from jax.experimental import pallas as pl
from jax.experimental.pallas import tpu as pltpu
from jax.experimental.pallas import tpu_sc as plsc

BATCH = 16384
EMBED = 32
NROWS = 1000000
NC = 2
NS = 16
NW = NC * NS
ROWS_PER_W = BATCH // NW
CHUNK = 128
NCHUNK = ROWS_PER_W // CHUNK

BLK = 49152
NBLK = -(-NROWS // BLK)
SPAD = NBLK * BLK


def _tc_score_body(w_ref, u_ref, a_ref, su_ref, sa_ref):
    wu = w_ref[0, :EMBED].reshape(EMBED, 1)
    wa = w_ref[0, EMBED:].reshape(EMBED, 1)
    su_ref[...] = jnp.sum(u_ref[...] * wu, axis=0, keepdims=True)
    sa_ref[...] = jnp.sum(a_ref[...] * wa, axis=0, keepdims=True)


_tc_score = pl.pallas_call(
    _tc_score_body,
    grid=(NBLK,),
    in_specs=[
        pl.BlockSpec((1, 2 * EMBED), lambda i: (0, 0)),
        pl.BlockSpec((EMBED, BLK), lambda i: (0, i)),
        pl.BlockSpec((EMBED, BLK), lambda i: (0, i)),
    ],
    out_specs=[
        pl.BlockSpec((1, BLK), lambda i: (0, i)),
        pl.BlockSpec((1, BLK), lambda i: (0, i)),
    ],
    out_shape=[
        jax.ShapeDtypeStruct((1, SPAD), jnp.float32),
        jax.ShapeDtypeStruct((1, SPAD), jnp.float32),
    ],
    compiler_params=pltpu.CompilerParams(
        dimension_semantics=("arbitrary",)),
)


@functools.partial(
    pl.kernel,
    out_type=jax.ShapeDtypeStruct((BATCH,), jnp.float32),
    mesh=plsc.VectorSubcoreMesh(core_axis_name="c", subcore_axis_name="s"),
    scratch_types=[
        pltpu.VMEM((NCHUNK, CHUNK), jnp.int32),
        pltpu.VMEM((NCHUNK, CHUNK), jnp.int32),
        pltpu.VMEM((ROWS_PER_W,), jnp.float32),
        pltpu.VMEM((ROWS_PER_W,), jnp.float32),
        pltpu.VMEM((16,), jnp.float32),
        pltpu.VMEM((ROWS_PER_W,), jnp.float32),
        pltpu.SemaphoreType.DMA,
    ],
    compiler_params=pltpu.CompilerParams(
        needs_layout_passes=False, use_tc_tiling_on_sc=False),
)
def _sc_combine_kernel(uids, aids, su, sa, fcb, out,
                       uidx_v, aidx_v, su_v, sa_v, b_v, out_v, sem):
    wid = lax.axis_index("s") * NC + lax.axis_index("c")
    base = wid * ROWS_PER_W

    pltpu.sync_copy(uids.at[wid], uidx_v)
    pltpu.sync_copy(aids.at[wid], aidx_v)
    pltpu.sync_copy(fcb, b_v.at[pl.ds(0, 1)])

    copies = []
    for j in range(NCHUNK):
        copies.append(pltpu.async_copy(
            su.at[uidx_v.at[j]], su_v.at[pl.ds(j * CHUNK, CHUNK)], sem))
        copies.append(pltpu.async_copy(
            sa.at[aidx_v.at[j]], sa_v.at[pl.ds(j * CHUNK, CHUNK)], sem))
    for c in copies:
        c.wait()

    bias_vec = b_v[...].at[jnp.zeros((16,), jnp.int32)].get(
        mode="promise_in_bounds")
    for g in range(ROWS_PER_W // 16):
        sl = pl.ds(g * 16, 16)
        out_v[sl] = su_v[sl] + sa_v[sl] + bias_vec

    pltpu.sync_copy(out_v, out.at[pl.ds(base, ROWS_PER_W)])


def kernel(user_ids, article_ids, user_table, article_table, fc_w, fc_b):
    uids = user_ids.astype(jnp.int32).reshape(NW, NCHUNK, CHUNK)
    aids = article_ids.astype(jnp.int32).reshape(NW, NCHUNK, CHUNK)
    su2, sa2 = _tc_score(fc_w, user_table.T, article_table.T)
    su = su2.reshape(SPAD)
    sa = sa2.reshape(SPAD)
    out = _sc_combine_kernel(uids, aids, su, sa, fc_b)
    return out.reshape(BATCH, 1)


# speedup vs baseline: 1.0201x; 1.0201x over previous
import functools

import jax
import jax.numpy as jnp
from jax import lax
from jax.experimental import pallas as pl
from jax.experimental.pallas import tpu as pltpu
from jax.experimental.pallas import tpu_sc as plsc

BATCH = 16384
EMBED = 32
NROWS = 1000000
NC = 2
NS = 16
NW = NC * NS
ROWS_PER_W = BATCH // NW
CHUNK = 128
NCHUNK = ROWS_PER_W // CHUNK

BLK = 32768
NBLK = -(-NROWS // BLK)
SPAD = NBLK * BLK


def _tc_score_body(w_ref, u_ref, a_ref, su_ref, sa_ref):
    wu = w_ref[0, :EMBED].reshape(EMBED, 1)
    wa = w_ref[0, EMBED:].reshape(EMBED, 1)
    su_ref[...] = jnp.sum(u_ref[...] * wu, axis=0, keepdims=True)
    sa_ref[...] = jnp.sum(a_ref[...] * wa, axis=0, keepdims=True)


_tc_score = pl.pallas_call(
    _tc_score_body,
    grid=(NBLK,),
    in_specs=[
        pl.BlockSpec((1, 2 * EMBED), lambda i: (0, 0)),
        pl.BlockSpec((EMBED, BLK), lambda i: (0, i)),
        pl.BlockSpec((EMBED, BLK), lambda i: (0, i)),
    ],
    out_specs=[
        pl.BlockSpec((1, BLK), lambda i: (0, i)),
        pl.BlockSpec((1, BLK), lambda i: (0, i)),
    ],
    out_shape=[
        jax.ShapeDtypeStruct((1, SPAD), jnp.float32),
        jax.ShapeDtypeStruct((1, SPAD), jnp.float32),
    ],
    compiler_params=pltpu.CompilerParams(
        dimension_semantics=("arbitrary",)),
)


@functools.partial(
    pl.kernel,
    out_type=jax.ShapeDtypeStruct((BATCH,), jnp.float32),
    mesh=plsc.VectorSubcoreMesh(core_axis_name="c", subcore_axis_name="s"),
    scratch_types=[
        pltpu.VMEM((NCHUNK, CHUNK), jnp.int32),
        pltpu.VMEM((NCHUNK, CHUNK), jnp.int32),
        pltpu.VMEM((ROWS_PER_W,), jnp.float32),
        pltpu.VMEM((ROWS_PER_W,), jnp.float32),
        pltpu.VMEM((16,), jnp.float32),
        pltpu.VMEM((ROWS_PER_W,), jnp.float32),
        pltpu.SemaphoreType.DMA,
    ],
    compiler_params=pltpu.CompilerParams(
        needs_layout_passes=False, use_tc_tiling_on_sc=False),
)
def _sc_combine_kernel(uids, aids, su, sa, fcb, out,
                       uidx_v, aidx_v, su_v, sa_v, b_v, out_v, sem):
    wid = lax.axis_index("s") * NC + lax.axis_index("c")
    base = wid * ROWS_PER_W

    pltpu.sync_copy(uids.at[wid], uidx_v)
    pltpu.sync_copy(aids.at[wid], aidx_v)
    pltpu.sync_copy(fcb, b_v.at[pl.ds(0, 1)])

    copies = []
    for j in range(NCHUNK):
        copies.append(pltpu.async_copy(
            su.at[uidx_v.at[j]], su_v.at[pl.ds(j * CHUNK, CHUNK)], sem))
        copies.append(pltpu.async_copy(
            sa.at[aidx_v.at[j]], sa_v.at[pl.ds(j * CHUNK, CHUNK)], sem))
    for c in copies:
        c.wait()

    bias_vec = b_v[...].at[jnp.zeros((16,), jnp.int32)].get(
        mode="promise_in_bounds")
    for g in range(ROWS_PER_W // 16):
        sl = pl.ds(g * 16, 16)
        out_v[sl] = su_v[sl] + sa_v[sl] + bias_vec

    pltpu.sync_copy(out_v, out.at[pl.ds(base, ROWS_PER_W)])


def kernel(user_ids, article_ids, user_table, article_table, fc_w, fc_b):
    uids = user_ids.astype(jnp.int32).reshape(NW, NCHUNK, CHUNK)
    aids = article_ids.astype(jnp.int32).reshape(NW, NCHUNK, CHUNK)
    su2, sa2 = _tc_score(fc_w, user_table.T, article_table.T)
    su = su2.reshape(SPAD)
    sa = sa2.reshape(SPAD)
    out = _sc_combine_kernel(uids, aids, su, sa, fc_b)
    return out.reshape(BATCH, 1)
